# baseline (speedup 1.0000x reference)
import functools

import jax
import jax.numpy as jnp
from jax import lax
from jax.experimental import pallas as pl
from jax.experimental.pallas import tpu as pltpu
from jax.experimental.pallas import tpu_sc as plsc

N = 262144
J = 8
K = 64
L = 16
NW = 32
NCHUNK = 4
NC_ROWS = N // NCHUNK
ROWS_W = NC_ROWS // NW
R = 128
T = ROWS_W // R
H = 8
KG = K // L
U = 16

OFF_W = (R + H) * J + L
DIA_W = R + L

_mesh = plsc.VectorSubcoreMesh(core_axis_name="c", subcore_axis_name="s")


@functools.partial(
    pl.kernel,
    mesh=_mesh,
    out_type=jax.ShapeDtypeStruct((NC_ROWS, K), jnp.float32),
    scratch_types=[
        pltpu.VMEM((R + 2 * H, K), jnp.float32),
        pltpu.VMEM((R + 2 * H, K), jnp.float32),
        pltpu.VMEM((OFF_W,), jnp.float32),
        pltpu.VMEM((OFF_W,), jnp.float32),
        pltpu.VMEM((DIA_W,), jnp.float32),
        pltpu.VMEM((DIA_W,), jnp.float32),
        pltpu.VMEM((R, K), jnp.float32),
        pltpu.VMEM((R, K), jnp.float32),
        pltpu.SemaphoreType.DMA,
        pltpu.SemaphoreType.DMA,
        pltpu.SemaphoreType.DMA,
        pltpu.SemaphoreType.DMA,
    ],
    compiler_params=pltpu.CompilerParams(use_tc_tiling_on_sc=True),
)
def _banded_sc(diag_hbm, off_hbm, other_hbm, out_hbm,
               oth_a, oth_b, off_a, off_b, dia_a, dia_b, out_a, out_b,
               sin_a, sin_b, sout_a, sout_b):
    cid = lax.axis_index("c")
    sid = lax.axis_index("s")
    wid = sid * 2 + cid
    wbase = wid * ROWS_W

    def start_in(t, oth_v, off_v, dia_v, sem):
        g0 = wbase + t * R
        pltpu.async_copy(other_hbm.at[pl.ds(g0, R + 2 * H)], oth_v, sem)
        pltpu.async_copy(off_hbm.at[pl.ds(g0 * J, (R + H) * J)],
                         off_v.at[pl.ds(0, (R + H) * J)], sem)
        pltpu.async_copy(diag_hbm.at[pl.ds(g0, R)], dia_v.at[pl.ds(0, R)], sem)

    def wait_in(t, oth_v, off_v, dia_v, sem):
        pltpu.make_async_copy(other_hbm.at[pl.ds(0, R + 2 * H)],
                              oth_v, sem).wait()
        pltpu.make_async_copy(off_hbm.at[pl.ds(0, (R + H) * J)],
                              off_v.at[pl.ds(0, (R + H) * J)], sem).wait()
        pltpu.make_async_copy(diag_hbm.at[pl.ds(0, R)],
                              dia_v.at[pl.ds(0, R)], sem).wait()

    def wait_out(out_v, sem):
        pltpu.make_async_copy(out_hbm.at[pl.ds(0, R)],
                              out_v, sem).wait()

    def compute(t, oth_v, off_v, dia_v, out_v):
        def winload(row, kg):
            return oth_v[row, pl.ds(kg * L, L)]

        def run_pass(kg0):
            init = tuple(winload(i, kg0) for i in range(U)) + \
                   tuple(winload(i, kg0 + 1) for i in range(U))

            def blk_body(b, carry):
                w0 = list(carry[:U])
                w1 = list(carry[U:])
                r0 = b * U
                dv = dia_v[pl.ds(r0, L)]
                lows = [off_v[pl.ds(8 * r0 + 7 + 16 * m, L)] for m in range(11)]
                uvecs = [off_v[pl.ds((r0 + 2 * h + H) * J, L)]
                         for h in range(U // 2)]
                for rr in range(U):
                    r = r0 + rr
                    d = dv[rr]
                    uv = uvecs[rr // 2]
                    ub = 8 * (rr % 2)
                    cu = [uv[ub + j] for j in range(J)]
                    cl = []
                    for j in range(J):
                        q = 8 * rr + 56 - 7 * j
                        cl.append(lows[(q - 7) // 16][(q - 7) % 16])
                    f0 = w0 + [winload(r + U, kg0)]
                    f1 = w1 + [winload(r + U, kg0 + 1)]
                    acc0 = d * f0[8]
                    acc1 = d * f1[8]
                    for j in range(J):
                        acc0 = acc0 + cu[j] * f0[9 + j]
                        acc1 = acc1 + cu[j] * f1[9 + j]
                        acc0 = acc0 + cl[j] * f0[7 - j]
                        acc1 = acc1 + cl[j] * f1[7 - j]
                    out_v[r, pl.ds(kg0 * L, L)] = acc0
                    out_v[r, pl.ds((kg0 + 1) * L, L)] = acc1
                    w0 = f0[1:]
                    w1 = f1[1:]
                return tuple(w0 + w1)

            lax.fori_loop(0, R // U, blk_body, init)

        run_pass(0)
        run_pass(2)

    def start_out(t, out_v, sem):
        g0 = wbase + t * R
        pltpu.async_copy(out_v, out_hbm.at[pl.ds(g0, R)], sem)

    start_in(0, oth_a, off_a, dia_a, sin_a)

    def pipe_body(i, carry):
        ta = 2 * i
        tb = 2 * i + 1
        start_in(tb, oth_b, off_b, dia_b, sin_b)
        wait_in(ta, oth_a, off_a, dia_a, sin_a)

        @pl.when(i > 0)
        def _():
            wait_out(out_a, sout_a)

        compute(ta, oth_a, off_a, dia_a, out_a)
        start_out(ta, out_a, sout_a)

        @pl.when(i < T // 2 - 1)
        def _():
            start_in(ta + 2, oth_a, off_a, dia_a, sin_a)

        wait_in(tb, oth_b, off_b, dia_b, sin_b)

        @pl.when(i > 0)
        def _():
            wait_out(out_b, sout_b)

        compute(tb, oth_b, off_b, dia_b, out_b)
        start_out(tb, out_b, sout_b)
        return carry

    lax.fori_loop(0, T // 2, pipe_body, 0)
    wait_out(out_a, sout_a)
    wait_out(out_b, sout_b)


def kernel(diag, off_diags, other):
    zo = jnp.zeros((H, K), jnp.float32)
    zf = jnp.zeros((H, J), jnp.float32)
    outs = []
    for c in range(NCHUNK):
        c0 = c * NC_ROWS
        c1 = c0 + NC_ROWS
        if c == 0:
            oth_c = jnp.concatenate([zo, other[: c1 + H]], axis=0)
            off_c = jnp.concatenate([zf, off_diags[:c1]], axis=0)
        elif c == NCHUNK - 1:
            oth_c = jnp.concatenate([other[c0 - H:], zo], axis=0)
            off_c = off_diags[c0 - H: c1]
        else:
            oth_c = other[c0 - H: c1 + H]
            off_c = off_diags[c0 - H: c1]
        outs.append(_banded_sc(diag[c0:c1], off_c.reshape(-1), oth_c))
    return jnp.concatenate(outs, axis=0)


# speedup vs baseline: 1.3749x; 1.3749x over previous
import functools

import jax
import jax.numpy as jnp
from jax import lax
from jax.experimental import pallas as pl
from jax.experimental.pallas import tpu as pltpu
from jax.experimental.pallas import tpu_sc as plsc

N = 262144
J = 8
K = 64
L = 16
NW = 32
ROWS_W = N // NW
R = 128
T = ROWS_W // R
H = 8
KG = K // L
U = 16

OFFJ_W = 256
OFF_W = J * OFFJ_W
DIA_W = R + L

_mesh = plsc.VectorSubcoreMesh(core_axis_name="c", subcore_axis_name="s")


@functools.partial(
    pl.kernel,
    mesh=_mesh,
    out_type=jax.ShapeDtypeStruct((N, K), jnp.float32),
    scratch_types=[
        pltpu.VMEM((R + 2 * H, K), jnp.float32),
        pltpu.VMEM((R + 2 * H, K), jnp.float32),
        pltpu.VMEM((J, 2 * R), jnp.float32),
        pltpu.VMEM((J, 2 * R), jnp.float32),
        pltpu.VMEM((OFF_W,), jnp.float32),
        pltpu.VMEM((DIA_W,), jnp.float32),
        pltpu.VMEM((DIA_W,), jnp.float32),
        pltpu.VMEM((R, K), jnp.float32),
        pltpu.VMEM((R, K), jnp.float32),
        pltpu.SemaphoreType.DMA,
        pltpu.SemaphoreType.DMA,
        pltpu.SemaphoreType.DMA,
        pltpu.SemaphoreType.DMA,
    ],
    compiler_params=pltpu.CompilerParams(use_tc_tiling_on_sc=True),
)
def _banded_sc(diag_hbm, offt_hbm, other_hbm, out_hbm,
               oth_a, oth_b, off_a, off_b, off_f, dia_a, dia_b, out_a, out_b,
               sin_a, sin_b, sout_a, sout_b):
    cid = lax.axis_index("c")
    sid = lax.axis_index("s")
    wid = sid * 2 + cid
    wbase = wid * ROWS_W

    zero16 = jnp.zeros((L,), jnp.float32)

    def edge_preds(t):
        first = (wid == 0) & (t == 0)
        last = (wid == NW - 1) & (t == T - 1)
        return first, last, jnp.logical_not(first | last)

    def start_in(t, oth_v, off_v, dia_v, sem):
        g0 = wbase + t * R
        first, last, mid = edge_preds(t)

        @pl.when(first)
        def _():
            pltpu.async_copy(other_hbm.at[pl.ds(0, R + H)],
                             oth_v.at[pl.ds(H, R + H)], sem)
            pltpu.async_copy(offt_hbm.at[:, pl.ds(0, R)],
                             off_v.at[:, pl.ds(R, R)], sem)

        @pl.when(last)
        def _():
            pltpu.async_copy(other_hbm.at[pl.ds(g0 - H, R + H)],
                             oth_v.at[pl.ds(0, R + H)], sem)
            pltpu.async_copy(offt_hbm.at[:, pl.ds(g0 - R, 2 * R)],
                             off_v, sem)

        @pl.when(mid)
        def _():
            pltpu.async_copy(other_hbm.at[pl.ds(g0 - H, R + 2 * H)],
                             oth_v.at[pl.ds(0, R + 2 * H)], sem)
            pltpu.async_copy(offt_hbm.at[:, pl.ds(g0 - R, 2 * R)],
                             off_v, sem)

        pltpu.async_copy(diag_hbm.at[pl.ds(g0, R)], dia_v.at[pl.ds(0, R)], sem)

    def wait_in(t, oth_v, off_v, dia_v, sem):
        first, last, mid = edge_preds(t)

        @pl.when(first | last)
        def _():
            pltpu.make_async_copy(other_hbm.at[pl.ds(0, R + H)],
                                  oth_v.at[pl.ds(0, R + H)], sem).wait()

        @pl.when(mid)
        def _():
            pltpu.make_async_copy(other_hbm.at[pl.ds(0, R + 2 * H)],
                                  oth_v.at[pl.ds(0, R + 2 * H)], sem).wait()

        @pl.when(first)
        def _():
            pltpu.make_async_copy(offt_hbm.at[:, pl.ds(0, R)],
                                  off_v.at[:, pl.ds(0, R)], sem).wait()

        @pl.when(last | mid)
        def _():
            pltpu.make_async_copy(offt_hbm.at[:, pl.ds(0, 2 * R)],
                                  off_v, sem).wait()

        pltpu.make_async_copy(diag_hbm.at[pl.ds(0, R)],
                              dia_v.at[pl.ds(0, R)], sem).wait()

        @pl.when(first)
        def _():
            for rr in range(H):
                for c in range(KG):
                    oth_v[rr, pl.ds(c * L, L)] = zero16

        @pl.when(last)
        def _():
            for rr in range(H):
                for c in range(KG):
                    oth_v[R + H + rr, pl.ds(c * L, L)] = zero16

    def wait_out(out_v, sem):
        pltpu.make_async_copy(out_hbm.at[pl.ds(0, R)],
                              out_v, sem).wait()

    def compute(t, oth_v, off_v, dia_v, out_v):
        first, _, _ = edge_preds(t)

        for j in range(J):
            for m in range(2 * R // L):
                off_f[pl.ds(j * OFFJ_W + m * L, L)] = off_v[j, pl.ds(m * L, L)]

        @pl.when(first)
        def _():
            for j in range(J):
                off_f[pl.ds(j * OFFJ_W + R - L, L)] = zero16

        def winload(row, kg):
            return oth_v[row, pl.ds(kg * L, L)]

        def run_pass(kg0):
            init = tuple(winload(i, kg0) for i in range(U)) + \
                   tuple(winload(i, kg0 + 1) for i in range(U))

            def blk_body(b, carry):
                w0 = list(carry[:U])
                w1 = list(carry[U:])
                r0 = b * U
                dv = dia_v[pl.ds(r0, L)]
                upv = [off_f[pl.ds(j * OFFJ_W + R + r0, L)]
                       for j in range(J)]
                lov = [off_f[pl.ds(j * OFFJ_W + R + r0 - j - 1, L)]
                       for j in range(J)]
                for rr in range(U):
                    r = r0 + rr
                    d = dv[rr]
                    cu = [upv[j][rr] for j in range(J)]
                    cl = [lov[j][rr] for j in range(J)]
                    f0 = w0 + [winload(r + U, kg0)]
                    f1 = w1 + [winload(r + U, kg0 + 1)]
                    acc0 = d * f0[8]
                    acc1 = d * f1[8]
                    for j in range(J):
                        acc0 = acc0 + cu[j] * f0[9 + j]
                        acc1 = acc1 + cu[j] * f1[9 + j]
                        acc0 = acc0 + cl[j] * f0[7 - j]
                        acc1 = acc1 + cl[j] * f1[7 - j]
                    out_v[r, pl.ds(kg0 * L, L)] = acc0
                    out_v[r, pl.ds((kg0 + 1) * L, L)] = acc1
                    w0 = f0[1:]
                    w1 = f1[1:]
                return tuple(w0 + w1)

            lax.fori_loop(0, R // U, blk_body, init)

        run_pass(0)
        run_pass(2)

    def start_out(t, out_v, sem):
        g0 = wbase + t * R
        pltpu.async_copy(out_v, out_hbm.at[pl.ds(g0, R)], sem)

    start_in(0, oth_a, off_a, dia_a, sin_a)

    def pipe_body(i, carry):
        ta = 2 * i
        tb = 2 * i + 1
        start_in(tb, oth_b, off_b, dia_b, sin_b)
        wait_in(ta, oth_a, off_a, dia_a, sin_a)

        @pl.when(i > 0)
        def _():
            wait_out(out_a, sout_a)

        compute(ta, oth_a, off_a, dia_a, out_a)
        start_out(ta, out_a, sout_a)

        @pl.when(i < T // 2 - 1)
        def _():
            start_in(ta + 2, oth_a, off_a, dia_a, sin_a)

        wait_in(tb, oth_b, off_b, dia_b, sin_b)

        @pl.when(i > 0)
        def _():
            wait_out(out_b, sout_b)

        compute(tb, oth_b, off_b, dia_b, out_b)
        start_out(tb, out_b, sout_b)
        return carry

    lax.fori_loop(0, T // 2, pipe_body, 0)
    wait_out(out_a, sout_a)
    wait_out(out_b, sout_b)


def kernel(diag, off_diags, other):
    return _banded_sc(diag, off_diags.T, other)
